# baseline (speedup 1.0000x reference)
import functools

import jax
import jax.numpy as jnp
from jax import lax
from jax.experimental import pallas as pl
from jax.experimental.pallas import tpu as pltpu
from jax.experimental.pallas import tpu_sc as plsc

N = 10000
IN_F = 128
F = 128
K = 32
L = 16
NC, NS = 2, 16
NW = NC * NS
RPW = -(-N // NW)
NCHUNK = N // L
GC = 8
NG = -(-NCHUNK // GC)
NGB = -(-NG // L)
PAD = NG * GC * L - N
BSTRIDE = N + PAD
HM_SHIFT = 19
HM_SIZE = 2048
HR_SHIFT = 11
RB = 256
MM_BLOCK = 400


def _mm_body(x_ref, w_ref, b_ref, o_ref):
    o_ref[...] = lax.dot_general(
        x_ref[...], w_ref[...], (((1,), (1,)), ((), ())),
        preferred_element_type=jnp.float32) + b_ref[...]


_matmul = pl.pallas_call(
    _mm_body,
    out_shape=jax.ShapeDtypeStruct((N, F), jnp.float32),
    grid=(N // MM_BLOCK,),
    in_specs=[
        pl.BlockSpec((MM_BLOCK, IN_F), lambda i: (i, 0)),
        pl.BlockSpec((F, IN_F), lambda i: (0, 0)),
        pl.BlockSpec((1, F), lambda i: (0, 0)),
    ],
    out_specs=pl.BlockSpec((MM_BLOCK, F), lambda i: (i, 0)),
)


_mesh = plsc.VectorSubcoreMesh(core_axis_name="c", subcore_axis_name="s")


@functools.partial(
    pl.kernel,
    out_type=jax.ShapeDtypeStruct((N, F), jnp.float32),
    mesh=_mesh,
    compiler_params=pltpu.CompilerParams(needs_layout_passes=False),
    scratch_types=[
        pltpu.VMEM((2 * (N + PAD),), jnp.float32),
        pltpu.VMEM((NG + L,), jnp.float32),
        pltpu.VMEM((HM_SIZE + L,), jnp.int32),
        pltpu.VMEM((NCHUNK + L,), jnp.int32),
        pltpu.VMEM((N + L,), jnp.int32),
        pltpu.VMEM((N + L,), jnp.int32),
        pltpu.VMEM((N + L,), jnp.int32),
        pltpu.VMEM((K + L,), jnp.int32),
        pltpu.VMEM((K,), jnp.int32),
        pltpu.VMEM((K, F), jnp.float32),
        pltpu.VMEM((F,), jnp.float32),
        pltpu.SemaphoreType.DMA,
        pltpu.SemaphoreType.DMA((2,)),
    ],
)
def _sc_topk(adj_flat_hbm, h_hbm, out_hbm, rowbuf, maxbuf, hist, qid, cbits,
             cidx,
             eqidx, fpad, fidx, grows, orow, sem, rsem):
    wid = lax.axis_index("s") * NC + lax.axis_index("c")
    base = wid * RPW
    nrows = jnp.minimum(RPW, N - base)
    ones = jnp.ones((L,), jnp.int32)
    zeros = jnp.zeros((L,), jnp.int32)
    neg1 = jnp.full((L,), -1, jnp.int32)
    lanes = lax.iota(jnp.int32, L)
    last = lanes == (L - 1)

    def clear_hist(nbkt):
        def clr(j, carry):
            hist[pl.ds(j * L, L)] = zeros
            return carry
        lax.fori_loop(0, nbkt // L, clr, 0, unroll=4)

    def hist_scan(start_cnt, nbkt):
        def cond(st):
            _, cnt, _ = st
            return cnt < K

        def body(st):
            j, cnt, _ = st
            nj = j - 1
            v = hist[pl.ds(nj * L, L)]
            tot = plsc.cumsum(v)[L - 1]
            return nj, cnt + tot, cnt

        j, _, prev = lax.while_loop(
            cond, body, (jnp.int32(nbkt // L), start_cnt, start_cnt))
        v = hist[pl.ds(j * L, L)]
        cs = plsc.cumsum(v)
        tot = cs[L - 1]
        incl = (prev + tot) + (v - cs)
        w = jnp.where(incl >= K, lanes, neg1)
        lane = plsc.cummax(w)[L - 1]
        b = j * L + lane
        sel = jnp.where(lanes == lane, cs, zeros)
        above = prev + tot - plsc.cumsum(sel)[L - 1]
        return b, above

    fzeros = jnp.zeros((L,), jnp.float32)
    for t in range(PAD // L):
        rowbuf[pl.ds(N + t * L, L)] = fzeros
        rowbuf[pl.ds(BSTRIDE + N + t * L, L)] = fzeros

    pltpu.async_copy(adj_flat_hbm.at[pl.ds(base * N, N)],
                     rowbuf.at[pl.ds(0, N)], rsem.at[0])

    def row_body(i, carry):
        r = base + i
        bsel = i & 1
        roff = bsel * BSTRIDE
        pltpu.make_async_copy(adj_flat_hbm.at[pl.ds(r * N, N)],
                              rowbuf.at[pl.ds(roff, N)],
                              rsem.at[bsel]).wait()

        @pl.when(i + 1 < nrows)
        def _prefetch():
            pltpu.async_copy(adj_flat_hbm.at[pl.ds((r + 1) * N, N)],
                             rowbuf.at[pl.ds((1 - bsel) * BSTRIDE, N)],
                             rsem.at[1 - bsel])

        def pm(g, carry):
            c = g * GC
            v = rowbuf[pl.ds(roff + c * L, L)]
            for t in range(1, GC):
                v = jnp.maximum(v, rowbuf[pl.ds(roff + (c + t) * L, L)])
            mx = plsc.cummax(v)
            plsc.store_scatter(maxbuf, [lanes * 0 + g], mx, mask=last)
            return carry
        lax.fori_loop(0, NG, pm, 0, unroll=4)

        clear_hist(HM_SIZE)

        def hm(g, carry):
            bits = lax.bitcast_convert_type(maxbuf[pl.ds(g * L, L)],
                                            jnp.int32)
            valid = (lanes + g * L) < NG
            bkt = lax.shift_right_logical(bits, HM_SHIFT)
            plsc.addupdate_scatter(hist, [bkt], ones, mask=valid)
            return carry
        lax.fori_loop(0, NGB, hm, 0, unroll=4)
        bm, am = hist_scan(jnp.int32(0), HM_SIZE)

        clear_hist(RB)

        def hr(g, carry):
            bits = lax.bitcast_convert_type(maxbuf[pl.ds(g * L, L)],
                                            jnp.int32)
            valid = (lanes + g * L) < NG
            m = valid & (lax.shift_right_logical(bits, HM_SHIFT) == bm)
            bkt = lax.shift_right_logical(bits, HR_SHIFT) & (RB - 1)
            plsc.addupdate_scatter(hist, [bkt], ones, mask=m)
            return carry
        lax.fori_loop(0, NGB, hr, 0, unroll=4)
        bm2, _ = hist_scan(am, RB)
        thr = lax.shift_left(lax.shift_left(bm, 8) | bm2, HR_SHIFT)

        def qc(g, cnt):
            bits = lax.bitcast_convert_type(maxbuf[pl.ds(g * L, L)],
                                            jnp.int32)
            valid = (lanes + g * L) < NG
            m = valid & (bits >= thr)
            plsc.store_compressed(qid.at[pl.ds(cnt, L)], lanes + g * L,
                                  mask=m)
            return cnt + plsc.all_reduce_population_count(m)[0]
        nq = lax.fori_loop(0, NGB, qc, jnp.int32(0), unroll=2)

        def cc(q, cnt):
            c = qid[pl.ds(q, L)][0] * GC
            for t in range(GC):
                idx = lanes + (c + t) * L
                bits = lax.bitcast_convert_type(
                    rowbuf[pl.ds(roff + (c + t) * L, L)], jnp.int32)
                m = (bits >= thr) & (idx < N)
                plsc.store_compressed(cbits.at[pl.ds(cnt, L)], bits, mask=m)
                plsc.store_compressed(cidx.at[pl.ds(cnt, L)], idx, mask=m)
                cnt = cnt + plsc.all_reduce_population_count(m)[0]
            return cnt
        m_cnt = lax.fori_loop(0, nq, cc, jnp.int32(0))
        ncc = (m_cnt + L - 1) // L

        def cand_round(shift, prefix_shift, prefix, start_cnt):
            clear_hist(RB)

            def hb(c, carry):
                bits = cbits[pl.ds(c * L, L)]
                m = (lanes + c * L) < m_cnt
                if prefix_shift is not None:
                    m = m & (lax.shift_right_logical(bits, prefix_shift)
                             == prefix)
                bkt = lax.shift_right_logical(bits, shift) & (RB - 1)
                plsc.addupdate_scatter(hist, [bkt], ones, mask=m)
                return carry
            lax.fori_loop(0, ncc, hb, 0)
            return hist_scan(start_cnt, RB)

        b0, c0 = cand_round(24, None, None, jnp.int32(0))
        b1, c1 = cand_round(16, 24, b0, c0)
        p2 = lax.shift_left(b0, 8) | b1
        b2, c2 = cand_round(8, 16, p2, c1)
        p3 = lax.shift_left(p2, 8) | b2
        b3, c3 = cand_round(0, 8, p3, c2)
        tbits = lax.shift_left(p3, 8) | b3
        n_tie = K - c3

        def fc(c, carry):
            ngt_vec, neq_vec = carry
            bits = cbits[pl.ds(c * L, L)]
            iv = cidx[pl.ds(c * L, L)]
            valid = (lanes + c * L) < m_cnt
            mgt = valid & (bits > tbits)
            meq = valid & (bits == tbits)
            gi = jnp.where(mgt, 1, 0)
            ei = jnp.where(meq, 1, 0)
            gpos = ngt_vec + plsc.cumsum(gi) - gi
            epos = neq_vec + plsc.cumsum(ei) - ei
            plsc.store_scatter(fpad, [gpos], iv, mask=mgt)
            plsc.store_scatter(eqidx, [epos], iv, mask=meq)
            ngt_vec = ngt_vec + plsc.all_reduce_population_count(mgt)
            neq_vec = neq_vec + plsc.all_reduce_population_count(meq)
            return ngt_vec, neq_vec
        lax.fori_loop(0, ncc, fc, (zeros, zeros))

        e0 = eqidx[pl.ds(0, L)]
        plsc.store_compressed(fpad.at[pl.ds(c3, L)], e0,
                              mask=lanes < jnp.minimum(n_tie, L))
        e1 = eqidx[pl.ds(L, L)]
        plsc.store_compressed(
            fpad.at[pl.ds(c3 + jnp.minimum(n_tie, L), L)], e1,
            mask=lanes < (n_tie - L))

        fidx[pl.ds(0, L)] = fpad[pl.ds(0, L)]
        fidx[pl.ds(L, L)] = fpad[pl.ds(L, L)]

        pltpu.async_copy(h_hbm.at[fidx], grows, sem).wait()

        def mx(j, accs):
            return tuple(
                jnp.maximum(a, grows[j, pl.ds(f * L, L)])
                for f, a in enumerate(accs))
        accs = lax.fori_loop(
            0, K, mx,
            tuple(jnp.full((L,), -jnp.inf, jnp.float32)
                  for _ in range(F // L)))
        for f in range(F // L):
            orow[pl.ds(f * L, L)] = accs[f]
        pltpu.sync_copy(orow, out_hbm.at[r])
        return carry

    lax.fori_loop(0, nrows, row_body, 0)


def kernel(x, adj, W, b):
    h = _matmul(x, W, b.reshape(1, F))
    return _sc_topk(adj.reshape(-1), h)


# speedup vs baseline: 1.0739x; 1.0739x over previous
import functools

import jax
import jax.numpy as jnp
from jax import lax
from jax.experimental import pallas as pl
from jax.experimental.pallas import tpu as pltpu
from jax.experimental.pallas import tpu_sc as plsc

N = 10000
IN_F = 128
F = 128
K = 32
L = 16
NC, NS = 2, 16
NW = NC * NS
RPW = -(-N // NW)
NCHUNK = N // L
GC = 4
NG = -(-NCHUNK // GC)
NGB = -(-NG // L)
PAD = NG * GC * L - N
BSTRIDE = N + PAD
HM_SHIFT = 19
HM_SIZE = 2048
HR_SHIFT = 11
RB = 256
MM_BLOCK = 400


def _mm_body(x_ref, w_ref, b_ref, o_ref):
    o_ref[...] = lax.dot_general(
        x_ref[...], w_ref[...], (((1,), (1,)), ((), ())),
        preferred_element_type=jnp.float32) + b_ref[...]


_matmul = pl.pallas_call(
    _mm_body,
    out_shape=jax.ShapeDtypeStruct((N, F), jnp.float32),
    grid=(N // MM_BLOCK,),
    in_specs=[
        pl.BlockSpec((MM_BLOCK, IN_F), lambda i: (i, 0)),
        pl.BlockSpec((F, IN_F), lambda i: (0, 0)),
        pl.BlockSpec((1, F), lambda i: (0, 0)),
    ],
    out_specs=pl.BlockSpec((MM_BLOCK, F), lambda i: (i, 0)),
)


_mesh = plsc.VectorSubcoreMesh(core_axis_name="c", subcore_axis_name="s")


@functools.partial(
    pl.kernel,
    out_type=jax.ShapeDtypeStruct((N, F), jnp.float32),
    mesh=_mesh,
    compiler_params=pltpu.CompilerParams(needs_layout_passes=False),
    scratch_types=[
        pltpu.VMEM((2 * (N + PAD),), jnp.float32),
        pltpu.VMEM((NG + L,), jnp.float32),
        pltpu.VMEM((HM_SIZE + L,), jnp.int32),
        pltpu.VMEM((NCHUNK + L,), jnp.int32),
        pltpu.VMEM((N + L,), jnp.int32),
        pltpu.VMEM((N + L,), jnp.int32),
        pltpu.VMEM((N + L,), jnp.int32),
        pltpu.VMEM((K + L,), jnp.int32),
        pltpu.VMEM((K,), jnp.int32),
        pltpu.VMEM((K, F), jnp.float32),
        pltpu.VMEM((F,), jnp.float32),
        pltpu.SemaphoreType.DMA,
        pltpu.SemaphoreType.DMA((2,)),
    ],
)
def _sc_topk(adj_flat_hbm, h_hbm, out_hbm, rowbuf, maxbuf, hist, qid, cbits,
             cidx,
             eqidx, fpad, fidx, grows, orow, sem, rsem):
    wid = lax.axis_index("s") * NC + lax.axis_index("c")
    base = wid * RPW
    nrows = jnp.minimum(RPW, N - base)
    ones = jnp.ones((L,), jnp.int32)
    zeros = jnp.zeros((L,), jnp.int32)
    neg1 = jnp.full((L,), -1, jnp.int32)
    lanes = lax.iota(jnp.int32, L)
    last = lanes == (L - 1)

    def clear_hist(nbkt):
        def clr(j, carry):
            hist[pl.ds(j * L, L)] = zeros
            return carry
        lax.fori_loop(0, nbkt // L, clr, 0, unroll=4)

    def hist_scan(start_cnt, nbkt):
        def cond(st):
            _, cnt, _ = st
            return cnt < K

        def body(st):
            j, cnt, _ = st
            nj = j - 1
            v = hist[pl.ds(nj * L, L)]
            tot = plsc.cumsum(v)[L - 1]
            return nj, cnt + tot, cnt

        j, _, prev = lax.while_loop(
            cond, body, (jnp.int32(nbkt // L), start_cnt, start_cnt))
        v = hist[pl.ds(j * L, L)]
        cs = plsc.cumsum(v)
        tot = cs[L - 1]
        incl = (prev + tot) + (v - cs)
        w = jnp.where(incl >= K, lanes, neg1)
        lane = plsc.cummax(w)[L - 1]
        b = j * L + lane
        sel = jnp.where(lanes == lane, cs, zeros)
        above = prev + tot - plsc.cumsum(sel)[L - 1]
        return b, above

    fzeros = jnp.zeros((L,), jnp.float32)
    for t in range(PAD // L):
        rowbuf[pl.ds(N + t * L, L)] = fzeros
        rowbuf[pl.ds(BSTRIDE + N + t * L, L)] = fzeros

    pltpu.async_copy(adj_flat_hbm.at[pl.ds(base * N, N)],
                     rowbuf.at[pl.ds(0, N)], rsem.at[0])

    def row_body(i, carry):
        r = base + i
        bsel = i & 1
        roff = bsel * BSTRIDE
        pltpu.make_async_copy(adj_flat_hbm.at[pl.ds(r * N, N)],
                              rowbuf.at[pl.ds(roff, N)],
                              rsem.at[bsel]).wait()

        @pl.when(i + 1 < nrows)
        def _prefetch():
            pltpu.async_copy(adj_flat_hbm.at[pl.ds((r + 1) * N, N)],
                             rowbuf.at[pl.ds((1 - bsel) * BSTRIDE, N)],
                             rsem.at[1 - bsel])

        def pm(g, carry):
            c = g * GC
            v = rowbuf[pl.ds(roff + c * L, L)]
            for t in range(1, GC):
                v = jnp.maximum(v, rowbuf[pl.ds(roff + (c + t) * L, L)])
            mx = plsc.cummax(v)
            plsc.store_scatter(maxbuf, [lanes * 0 + g], mx, mask=last)
            return carry
        lax.fori_loop(0, NG, pm, 0, unroll=8)

        clear_hist(HM_SIZE)

        def hm(g, carry):
            bits = lax.bitcast_convert_type(maxbuf[pl.ds(g * L, L)],
                                            jnp.int32)
            valid = (lanes + g * L) < NG
            bkt = lax.shift_right_logical(bits, HM_SHIFT)
            plsc.addupdate_scatter(hist, [bkt], ones, mask=valid)
            return carry
        lax.fori_loop(0, NGB, hm, 0, unroll=4)
        bm, am = hist_scan(jnp.int32(0), HM_SIZE)

        clear_hist(RB)

        def hr(g, carry):
            bits = lax.bitcast_convert_type(maxbuf[pl.ds(g * L, L)],
                                            jnp.int32)
            valid = (lanes + g * L) < NG
            m = valid & (lax.shift_right_logical(bits, HM_SHIFT) == bm)
            bkt = lax.shift_right_logical(bits, HR_SHIFT) & (RB - 1)
            plsc.addupdate_scatter(hist, [bkt], ones, mask=m)
            return carry
        lax.fori_loop(0, NGB, hr, 0, unroll=4)
        bm2, _ = hist_scan(am, RB)
        thr = lax.shift_left(lax.shift_left(bm, 8) | bm2, HR_SHIFT)

        def qc(g, cnt):
            bits = lax.bitcast_convert_type(maxbuf[pl.ds(g * L, L)],
                                            jnp.int32)
            valid = (lanes + g * L) < NG
            m = valid & (bits >= thr)
            plsc.store_compressed(qid.at[pl.ds(cnt, L)], lanes + g * L,
                                  mask=m)
            return cnt + plsc.all_reduce_population_count(m)[0]
        nq = lax.fori_loop(0, NGB, qc, jnp.int32(0), unroll=2)

        def cc(q, cnt):
            c = qid[pl.ds(q, L)][0] * GC
            for t in range(GC):
                idx = lanes + (c + t) * L
                bits = lax.bitcast_convert_type(
                    rowbuf[pl.ds(roff + (c + t) * L, L)], jnp.int32)
                m = (bits >= thr) & (idx < N)
                plsc.store_compressed(cbits.at[pl.ds(cnt, L)], bits, mask=m)
                plsc.store_compressed(cidx.at[pl.ds(cnt, L)], idx, mask=m)
                cnt = cnt + plsc.all_reduce_population_count(m)[0]
            return cnt
        m_cnt = lax.fori_loop(0, nq, cc, jnp.int32(0))
        ncc = (m_cnt + L - 1) // L

        def cand_round(shift, prefix_shift, prefix, start_cnt):
            clear_hist(RB)

            def hb(c, carry):
                bits = cbits[pl.ds(c * L, L)]
                m = (lanes + c * L) < m_cnt
                if prefix_shift is not None:
                    m = m & (lax.shift_right_logical(bits, prefix_shift)
                             == prefix)
                bkt = lax.shift_right_logical(bits, shift) & (RB - 1)
                plsc.addupdate_scatter(hist, [bkt], ones, mask=m)
                return carry
            lax.fori_loop(0, ncc, hb, 0)
            return hist_scan(start_cnt, RB)

        b0, c0 = cand_round(24, None, None, jnp.int32(0))
        b1, c1 = cand_round(16, 24, b0, c0)
        p2 = lax.shift_left(b0, 8) | b1
        b2, c2 = cand_round(8, 16, p2, c1)
        p3 = lax.shift_left(p2, 8) | b2
        b3, c3 = cand_round(0, 8, p3, c2)
        tbits = lax.shift_left(p3, 8) | b3
        n_tie = K - c3

        def fc(c, carry):
            ngt_vec, neq_vec = carry
            bits = cbits[pl.ds(c * L, L)]
            iv = cidx[pl.ds(c * L, L)]
            valid = (lanes + c * L) < m_cnt
            mgt = valid & (bits > tbits)
            meq = valid & (bits == tbits)
            gi = jnp.where(mgt, 1, 0)
            ei = jnp.where(meq, 1, 0)
            gpos = ngt_vec + plsc.cumsum(gi) - gi
            epos = neq_vec + plsc.cumsum(ei) - ei
            plsc.store_scatter(fpad, [gpos], iv, mask=mgt)
            plsc.store_scatter(eqidx, [epos], iv, mask=meq)
            ngt_vec = ngt_vec + plsc.all_reduce_population_count(mgt)
            neq_vec = neq_vec + plsc.all_reduce_population_count(meq)
            return ngt_vec, neq_vec
        lax.fori_loop(0, ncc, fc, (zeros, zeros))

        e0 = eqidx[pl.ds(0, L)]
        plsc.store_compressed(fpad.at[pl.ds(c3, L)], e0,
                              mask=lanes < jnp.minimum(n_tie, L))
        e1 = eqidx[pl.ds(L, L)]
        plsc.store_compressed(
            fpad.at[pl.ds(c3 + jnp.minimum(n_tie, L), L)], e1,
            mask=lanes < (n_tie - L))

        fidx[pl.ds(0, L)] = fpad[pl.ds(0, L)]
        fidx[pl.ds(L, L)] = fpad[pl.ds(L, L)]

        pltpu.async_copy(h_hbm.at[fidx], grows, sem).wait()

        def mx(j, accs):
            return tuple(
                jnp.maximum(a, grows[j, pl.ds(f * L, L)])
                for f, a in enumerate(accs))
        accs = lax.fori_loop(
            0, K, mx,
            tuple(jnp.full((L,), -jnp.inf, jnp.float32)
                  for _ in range(F // L)))
        for f in range(F // L):
            orow[pl.ds(f * L, L)] = accs[f]
        pltpu.sync_copy(orow, out_hbm.at[r])
        return carry

    lax.fori_loop(0, nrows, row_body, 0)


def kernel(x, adj, W, b):
    h = _matmul(x, W, b.reshape(1, F))
    return _sc_topk(adj.reshape(-1), h)


# speedup vs baseline: 1.2060x; 1.1230x over previous
import functools

import jax
import jax.numpy as jnp
from jax import lax
from jax.experimental import pallas as pl
from jax.experimental.pallas import tpu as pltpu
from jax.experimental.pallas import tpu_sc as plsc

N = 10000
IN_F = 128
F = 128
K = 32
L = 16
NC, NS = 2, 16
NW = NC * NS
RPW = -(-N // NW)
NCHUNK = N // L
GC = 4
NG = -(-NCHUNK // GC)
NGB = -(-NG // L)
PAD = NG * GC * L - N
BSTRIDE = N + PAD
HM_SHIFT = 19
HM_SIZE = 2048
HR_SHIFT = 11
RB = 256
MM_BLOCK = 400


def _mm_body(x_ref, w_ref, b_ref, o_ref):
    o_ref[...] = lax.dot_general(
        x_ref[...], w_ref[...], (((1,), (1,)), ((), ())),
        preferred_element_type=jnp.float32) + b_ref[...]


_matmul = pl.pallas_call(
    _mm_body,
    out_shape=jax.ShapeDtypeStruct((N, F), jnp.float32),
    grid=(N // MM_BLOCK,),
    in_specs=[
        pl.BlockSpec((MM_BLOCK, IN_F), lambda i: (i, 0)),
        pl.BlockSpec((F, IN_F), lambda i: (0, 0)),
        pl.BlockSpec((1, F), lambda i: (0, 0)),
    ],
    out_specs=pl.BlockSpec((MM_BLOCK, F), lambda i: (i, 0)),
)


_mesh = plsc.VectorSubcoreMesh(core_axis_name="c", subcore_axis_name="s")


@functools.partial(
    pl.kernel,
    out_type=jax.ShapeDtypeStruct((N, F), jnp.float32),
    mesh=_mesh,
    compiler_params=pltpu.CompilerParams(needs_layout_passes=False),
    scratch_types=[
        pltpu.VMEM((2 * (N + PAD),), jnp.float32),
        pltpu.VMEM((NG + L,), jnp.float32),
        pltpu.VMEM((HM_SIZE + L,), jnp.int32),
        pltpu.VMEM((NCHUNK + L,), jnp.int32),
        pltpu.VMEM((N + L,), jnp.int32),
        pltpu.VMEM((N + L,), jnp.int32),
        pltpu.VMEM((N + L,), jnp.int32),
        pltpu.VMEM((K + L,), jnp.int32),
        pltpu.VMEM((2, K), jnp.int32),
        pltpu.VMEM((2, K, F), jnp.float32),
        pltpu.VMEM((2, F), jnp.float32),
        pltpu.SemaphoreType.DMA((2,)),
        pltpu.SemaphoreType.DMA((2,)),
        pltpu.SemaphoreType.DMA((2,)),
    ],
)
def _sc_topk(adj_flat_hbm, h_hbm, out_hbm, rowbuf, maxbuf, hist, qid, cbits,
             cidx,
             eqidx, fpad, fidx, grows, orow, gsem, rsem, osem):
    wid = lax.axis_index("s") * NC + lax.axis_index("c")
    base = wid * RPW
    nrows = jnp.minimum(RPW, N - base)
    ones = jnp.ones((L,), jnp.int32)
    zeros = jnp.zeros((L,), jnp.int32)
    neg1 = jnp.full((L,), -1, jnp.int32)
    lanes = lax.iota(jnp.int32, L)
    last = lanes == (L - 1)

    def clear_hist(nbkt):
        def clr(j, carry):
            hist[pl.ds(j * L, L)] = zeros
            return carry
        lax.fori_loop(0, nbkt // L, clr, 0, unroll=4)

    def hist_scan(start_cnt, nbkt):
        def cond(st):
            _, cnt, _ = st
            return cnt < K

        def body(st):
            j, cnt, _ = st
            nj = j - 1
            v = hist[pl.ds(nj * L, L)]
            tot = plsc.cumsum(v)[L - 1]
            return nj, cnt + tot, cnt

        j, _, prev = lax.while_loop(
            cond, body, (jnp.int32(nbkt // L), start_cnt, start_cnt))
        v = hist[pl.ds(j * L, L)]
        cs = plsc.cumsum(v)
        tot = cs[L - 1]
        incl = (prev + tot) + (v - cs)
        w = jnp.where(incl >= K, lanes, neg1)
        lane = plsc.cummax(w)[L - 1]
        b = j * L + lane
        sel = jnp.where(lanes == lane, cs, zeros)
        above = prev + tot - plsc.cumsum(sel)[L - 1]
        return b, above

    fzeros = jnp.zeros((L,), jnp.float32)
    for t in range(PAD // L):
        rowbuf[pl.ds(N + t * L, L)] = fzeros
        rowbuf[pl.ds(BSTRIDE + N + t * L, L)] = fzeros

    pltpu.async_copy(adj_flat_hbm.at[pl.ds(base * N, N)],
                     rowbuf.at[pl.ds(0, N)], rsem.at[0])

    def finish(j):
        q = j & 1
        pltpu.make_async_copy(h_hbm.at[fidx.at[q]], grows.at[q],
                              gsem.at[q]).wait()

        @pl.when(j >= 2)
        def _wout():
            pltpu.make_async_copy(orow.at[q], out_hbm.at[base + j - 2],
                                  osem.at[q]).wait()

        def mx(k, accs):
            return tuple(
                jnp.maximum(a, grows[q, k, pl.ds(f * L, L)])
                for f, a in enumerate(accs))
        accs = lax.fori_loop(
            0, K, mx,
            tuple(jnp.full((L,), -jnp.inf, jnp.float32)
                  for _ in range(F // L)))
        for f in range(F // L):
            orow[q, pl.ds(f * L, L)] = accs[f]
        pltpu.async_copy(orow.at[q], out_hbm.at[base + j], osem.at[q])

    def row_body(i, carry):
        r = base + i
        bsel = i & 1
        roff = bsel * BSTRIDE
        pltpu.make_async_copy(adj_flat_hbm.at[pl.ds(r * N, N)],
                              rowbuf.at[pl.ds(roff, N)],
                              rsem.at[bsel]).wait()

        @pl.when(i + 1 < nrows)
        def _prefetch():
            pltpu.async_copy(adj_flat_hbm.at[pl.ds((r + 1) * N, N)],
                             rowbuf.at[pl.ds((1 - bsel) * BSTRIDE, N)],
                             rsem.at[1 - bsel])

        def pm(g, carry):
            c = g * GC
            v = rowbuf[pl.ds(roff + c * L, L)]
            for t in range(1, GC):
                v = jnp.maximum(v, rowbuf[pl.ds(roff + (c + t) * L, L)])
            mx = plsc.cummax(v)
            plsc.store_scatter(maxbuf, [lanes * 0 + g], mx, mask=last)
            return carry
        lax.fori_loop(0, NG, pm, 0, unroll=8)

        clear_hist(HM_SIZE)

        def hm(g, carry):
            bits = lax.bitcast_convert_type(maxbuf[pl.ds(g * L, L)],
                                            jnp.int32)
            valid = (lanes + g * L) < NG
            bkt = lax.shift_right_logical(bits, HM_SHIFT)
            plsc.addupdate_scatter(hist, [bkt], ones, mask=valid)
            return carry
        lax.fori_loop(0, NGB, hm, 0, unroll=4)
        bm, am = hist_scan(jnp.int32(0), HM_SIZE)

        clear_hist(RB)

        def hr(g, carry):
            bits = lax.bitcast_convert_type(maxbuf[pl.ds(g * L, L)],
                                            jnp.int32)
            valid = (lanes + g * L) < NG
            m = valid & (lax.shift_right_logical(bits, HM_SHIFT) == bm)
            bkt = lax.shift_right_logical(bits, HR_SHIFT) & (RB - 1)
            plsc.addupdate_scatter(hist, [bkt], ones, mask=m)
            return carry
        lax.fori_loop(0, NGB, hr, 0, unroll=4)
        bm2, _ = hist_scan(am, RB)
        thr = lax.shift_left(lax.shift_left(bm, 8) | bm2, HR_SHIFT)

        def qc(g, cnt):
            bits = lax.bitcast_convert_type(maxbuf[pl.ds(g * L, L)],
                                            jnp.int32)
            valid = (lanes + g * L) < NG
            m = valid & (bits >= thr)
            plsc.store_compressed(qid.at[pl.ds(cnt, L)], lanes + g * L,
                                  mask=m)
            return cnt + plsc.all_reduce_population_count(m)[0]
        nq = lax.fori_loop(0, NGB, qc, jnp.int32(0), unroll=2)

        def cc(q, cnt):
            c = qid[pl.ds(q, L)][0] * GC
            for t in range(GC):
                idx = lanes + (c + t) * L
                bits = lax.bitcast_convert_type(
                    rowbuf[pl.ds(roff + (c + t) * L, L)], jnp.int32)
                m = (bits >= thr) & (idx < N)
                plsc.store_compressed(cbits.at[pl.ds(cnt, L)], bits, mask=m)
                plsc.store_compressed(cidx.at[pl.ds(cnt, L)], idx, mask=m)
                cnt = cnt + plsc.all_reduce_population_count(m)[0]
            return cnt
        m_cnt = lax.fori_loop(0, nq, cc, jnp.int32(0))
        ncc = (m_cnt + L - 1) // L

        def cand_round(shift, prefix_shift, prefix, start_cnt):
            clear_hist(RB)

            def hb(c, carry):
                bits = cbits[pl.ds(c * L, L)]
                m = (lanes + c * L) < m_cnt
                if prefix_shift is not None:
                    m = m & (lax.shift_right_logical(bits, prefix_shift)
                             == prefix)
                bkt = lax.shift_right_logical(bits, shift) & (RB - 1)
                plsc.addupdate_scatter(hist, [bkt], ones, mask=m)
                return carry
            lax.fori_loop(0, ncc, hb, 0)
            return hist_scan(start_cnt, RB)

        b0, c0 = cand_round(24, None, None, jnp.int32(0))
        b1, c1 = cand_round(16, 24, b0, c0)
        p2 = lax.shift_left(b0, 8) | b1
        b2, c2 = cand_round(8, 16, p2, c1)
        p3 = lax.shift_left(p2, 8) | b2
        b3, c3 = cand_round(0, 8, p3, c2)
        tbits = lax.shift_left(p3, 8) | b3
        n_tie = K - c3

        def fc(c, carry):
            ngt_vec, neq_vec = carry
            bits = cbits[pl.ds(c * L, L)]
            iv = cidx[pl.ds(c * L, L)]
            valid = (lanes + c * L) < m_cnt
            mgt = valid & (bits > tbits)
            meq = valid & (bits == tbits)
            gi = jnp.where(mgt, 1, 0)
            ei = jnp.where(meq, 1, 0)
            gpos = ngt_vec + plsc.cumsum(gi) - gi
            epos = neq_vec + plsc.cumsum(ei) - ei
            plsc.store_scatter(fpad, [gpos], iv, mask=mgt)
            plsc.store_scatter(eqidx, [epos], iv, mask=meq)
            ngt_vec = ngt_vec + plsc.all_reduce_population_count(mgt)
            neq_vec = neq_vec + plsc.all_reduce_population_count(meq)
            return ngt_vec, neq_vec
        lax.fori_loop(0, ncc, fc, (zeros, zeros))

        e0 = eqidx[pl.ds(0, L)]
        plsc.store_compressed(fpad.at[pl.ds(c3, L)], e0,
                              mask=lanes < jnp.minimum(n_tie, L))
        e1 = eqidx[pl.ds(L, L)]
        plsc.store_compressed(
            fpad.at[pl.ds(c3 + jnp.minimum(n_tie, L), L)], e1,
            mask=lanes < (n_tie - L))

        fidx[bsel, pl.ds(0, L)] = fpad[pl.ds(0, L)]
        fidx[bsel, pl.ds(L, L)] = fpad[pl.ds(L, L)]

        pltpu.async_copy(h_hbm.at[fidx.at[bsel]], grows.at[bsel],
                         gsem.at[bsel])

        @pl.when(i > 0)
        def _prev():
            finish(i - 1)
        return carry

    lax.fori_loop(0, nrows, row_body, 0)
    finish(nrows - 1)
    pltpu.make_async_copy(orow.at[(nrows - 2) & 1],
                          out_hbm.at[base + nrows - 2],
                          osem.at[(nrows - 2) & 1]).wait()
    pltpu.make_async_copy(orow.at[(nrows - 1) & 1],
                          out_hbm.at[base + nrows - 1],
                          osem.at[(nrows - 1) & 1]).wait()


def kernel(x, adj, W, b):
    h = _matmul(x, W, b.reshape(1, F))
    return _sc_topk(adj.reshape(-1), h)
